# baseline (speedup 1.0000x reference)
import functools

import jax
import jax.numpy as jnp
from jax import lax
from jax.experimental import pallas as pl
from jax.experimental.pallas import tpu as pltpu
from jax.experimental.pallas import tpu_sc as plsc

EPS = 1e-12
GW = 32
NCHUNK = 2


def _sc_gather_chunk(word_emb, ids1d, tok_off, n_tok, d):
    info = plsc.get_sparse_core_info()
    nw = info.num_cores * info.num_subcores
    per_w = n_tok // nw
    ndma = per_w // GW
    mesh = plsc.VectorSubcoreMesh(core_axis_name="c", subcore_axis_name="s")

    @functools.partial(
        pl.kernel,
        out_type=jax.ShapeDtypeStruct((n_tok, d), jnp.float32),
        mesh=mesh,
        scratch_types=[
            pltpu.VMEM((per_w,), jnp.int32),
            pltpu.VMEM((2, GW, d), jnp.float32),
            pltpu.SemaphoreType.DMA((2,)),
            pltpu.SemaphoreType.DMA((2,)),
        ],
    )
    def k(table_hbm, idx_hbm, out_hbm, idx_v, buf, gsem, osem):
        wid = lax.axis_index("s") * info.num_cores + lax.axis_index("c")
        base = wid * per_w
        pltpu.sync_copy(idx_hbm.at[pl.ds(tok_off + base, per_w)], idx_v)
        handles_o = [None] * ndma
        for i in range(ndma):
            slot = i % 2
            if i >= 2:
                handles_o[i - 2].wait()
            g = pltpu.async_copy(
                table_hbm.at[idx_v.at[pl.ds(i * GW, GW)]],
                buf.at[slot],
                gsem.at[slot],
            )
            g.wait()
            handles_o[i] = pltpu.async_copy(
                buf.at[slot], out_hbm.at[pl.ds(base + i * GW, GW)], osem.at[slot]
            )
        for i in range(max(ndma - 2, 0), ndma):
            handles_o[i].wait()

    return k(word_emb, ids1d)


def _ln_body(g_ref, p_ref, w_ref, b_ref, o_ref):
    x = g_ref[...] + p_ref[...]
    u = jnp.mean(x, axis=1, keepdims=True)
    dlt = x - u
    v = jnp.mean(dlt * dlt, axis=1, keepdims=True)
    y = dlt * lax.rsqrt(v + EPS)
    y = y * w_ref[...] + b_ref[...]
    o_ref[0] = y.T


def _ln_body_acc(g_ref, p_ref, w_ref, b_ref, _buf_ref, o_ref):
    _ln_body(g_ref, p_ref, w_ref, b_ref, o_ref)


def _tc_chunk(gathered_c, pos_emb, w2d, b2d, buf, b_off, bpc, bsz, s, d):
    in_specs = [
        pl.BlockSpec((s, d), lambda b: (b, 0)),
        pl.BlockSpec((s, d), lambda b: (0, 0)),
        pl.BlockSpec((1, d), lambda b: (0, 0)),
        pl.BlockSpec((1, d), lambda b: (0, 0)),
    ]
    args = [gathered_c, pos_emb, w2d, b2d]
    if buf is None:
        body = _ln_body
        aliases = {}
    else:
        body = _ln_body_acc
        in_specs.append(pl.BlockSpec(memory_space=pl.ANY))
        args.append(buf)
        aliases = {4: 0}
    return pl.pallas_call(
        body,
        grid=(bpc,),
        in_specs=in_specs,
        out_specs=pl.BlockSpec(
            (1, d, s), lambda b, b_off=b_off: (b_off + b, 0, 0)
        ),
        out_shape=jax.ShapeDtypeStruct((bsz, d, s), jnp.float32),
        input_output_aliases=aliases,
        compiler_params=pltpu.CompilerParams(
            dimension_semantics=("arbitrary",),
        ),
    )(*args)


def kernel(input_ids, word_emb, pos_emb, ln_weight, ln_bias):
    bsz, s = input_ids.shape
    _, d = word_emb.shape
    bpc = bsz // NCHUNK
    ids1d = input_ids.astype(jnp.int32).reshape(bsz * s)
    w2d = ln_weight.reshape(1, d)
    b2d = ln_bias.reshape(1, d)
    gathered = [
        _sc_gather_chunk(word_emb, ids1d, c * bpc * s, bpc * s, d)
        for c in range(NCHUNK)
    ]
    buf = None
    for c in range(NCHUNK):
        buf = _tc_chunk(
            gathered[c], pos_emb, w2d, b2d, buf, c * bpc, bpc, bsz, s, d
        )
    return buf


# speedup vs baseline: 1.0109x; 1.0109x over previous
import functools

import jax
import jax.numpy as jnp
from jax import lax
from jax.experimental import pallas as pl
from jax.experimental.pallas import tpu as pltpu
from jax.experimental.pallas import tpu_sc as plsc

EPS = 1e-12
GW = 32
NCHUNK = 2


def _sc_gather_chunk(word_emb, ids1d, tok_off, n_tok, d):
    info = plsc.get_sparse_core_info()
    nw = info.num_cores * info.num_subcores
    per_w = n_tok // nw
    ndma = per_w // GW
    mesh = plsc.VectorSubcoreMesh(core_axis_name="c", subcore_axis_name="s")

    nslot = 3

    @functools.partial(
        pl.kernel,
        out_type=jax.ShapeDtypeStruct((n_tok, d), jnp.float32),
        mesh=mesh,
        scratch_types=[
            pltpu.VMEM((per_w,), jnp.int32),
            pltpu.VMEM((nslot, GW, d), jnp.float32),
            pltpu.SemaphoreType.DMA((nslot,)),
            pltpu.SemaphoreType.DMA((nslot,)),
        ],
    )
    def k(table_hbm, idx_hbm, out_hbm, idx_v, buf, gsem, osem):
        wid = lax.axis_index("s") * info.num_cores + lax.axis_index("c")
        base = wid * per_w
        pltpu.sync_copy(idx_hbm.at[pl.ds(tok_off + base, per_w)], idx_v)

        def start_gather(i):
            slot = i % nslot
            return pltpu.async_copy(
                table_hbm.at[idx_v.at[pl.ds(i * GW, GW)]],
                buf.at[slot],
                gsem.at[slot],
            )

        handles_g = [None] * ndma
        handles_o = [None] * ndma
        for i in range(min(2, ndma)):
            handles_g[i] = start_gather(i)
        for i in range(ndma):
            slot = i % nslot
            handles_g[i].wait()
            handles_o[i] = pltpu.async_copy(
                buf.at[slot], out_hbm.at[pl.ds(base + i * GW, GW)], osem.at[slot]
            )
            if i + 2 < ndma:
                if i - 1 >= 0:
                    handles_o[i - 1].wait()
                handles_g[i + 2] = start_gather(i + 2)
        for i in range(max(ndma - 2, 0), ndma):
            handles_o[i].wait()

    return k(word_emb, ids1d)


def _ln_body(g_ref, p_ref, w_ref, b_ref, o_ref):
    x = g_ref[...] + p_ref[...]
    u = jnp.mean(x, axis=1, keepdims=True)
    dlt = x - u
    v = jnp.mean(dlt * dlt, axis=1, keepdims=True)
    y = dlt * lax.rsqrt(v + EPS)
    y = y * w_ref[...] + b_ref[...]
    o_ref[0] = y.T


def _ln_body_acc(g_ref, p_ref, w_ref, b_ref, _buf_ref, o_ref):
    _ln_body(g_ref, p_ref, w_ref, b_ref, o_ref)


def _tc_chunk(gathered_c, pos_emb, w2d, b2d, buf, b_off, bpc, bsz, s, d):
    in_specs = [
        pl.BlockSpec((s, d), lambda b: (b, 0)),
        pl.BlockSpec((s, d), lambda b: (0, 0)),
        pl.BlockSpec((1, d), lambda b: (0, 0)),
        pl.BlockSpec((1, d), lambda b: (0, 0)),
    ]
    args = [gathered_c, pos_emb, w2d, b2d]
    if buf is None:
        body = _ln_body
        aliases = {}
    else:
        body = _ln_body_acc
        in_specs.append(pl.BlockSpec(memory_space=pl.ANY))
        args.append(buf)
        aliases = {4: 0}
    return pl.pallas_call(
        body,
        grid=(bpc,),
        in_specs=in_specs,
        out_specs=pl.BlockSpec(
            (1, d, s), lambda b, b_off=b_off: (b_off + b, 0, 0)
        ),
        out_shape=jax.ShapeDtypeStruct((bsz, d, s), jnp.float32),
        input_output_aliases=aliases,
        compiler_params=pltpu.CompilerParams(
            dimension_semantics=("arbitrary",),
        ),
    )(*args)


def kernel(input_ids, word_emb, pos_emb, ln_weight, ln_bias):
    bsz, s = input_ids.shape
    _, d = word_emb.shape
    bpc = bsz // NCHUNK
    ids1d = input_ids.astype(jnp.int32).reshape(bsz * s)
    w2d = ln_weight.reshape(1, d)
    b2d = ln_bias.reshape(1, d)
    gathered = [
        _sc_gather_chunk(word_emb, ids1d, c * bpc * s, bpc * s, d)
        for c in range(NCHUNK)
    ]
    buf = None
    for c in range(NCHUNK):
        buf = _tc_chunk(
            gathered[c], pos_emb, w2d, b2d, buf, c * bpc, bpc, bsz, s, d
        )
    return buf


# speedup vs baseline: 1.0235x; 1.0124x over previous
import functools

import jax
import jax.numpy as jnp
from jax import lax
from jax.experimental import pallas as pl
from jax.experimental.pallas import tpu as pltpu
from jax.experimental.pallas import tpu_sc as plsc

EPS = 1e-12
GW = 32
NCHUNK = 2


def _sc_gather_chunk(word_emb, ids1d, tok_off, n_tok, d):
    info = plsc.get_sparse_core_info()
    nw = info.num_cores * info.num_subcores
    per_w = n_tok // nw
    ndma = per_w // GW
    mesh = plsc.VectorSubcoreMesh(core_axis_name="c", subcore_axis_name="s")

    nslot = 3

    @functools.partial(
        pl.kernel,
        out_type=jax.ShapeDtypeStruct((n_tok, d), jnp.float32),
        mesh=mesh,
        scratch_types=[
            pltpu.VMEM((per_w,), jnp.int32),
            pltpu.VMEM((nslot, GW, d), jnp.float32),
            pltpu.SemaphoreType.DMA((nslot,)),
            pltpu.SemaphoreType.DMA((nslot,)),
        ],
    )
    def k(table_hbm, idx_hbm, out_hbm, idx_v, buf, gsem, osem):
        wid = lax.axis_index("s") * info.num_cores + lax.axis_index("c")
        base = wid * per_w
        pltpu.sync_copy(idx_hbm.at[pl.ds(tok_off + base, per_w)], idx_v)

        def start_gather(i):
            slot = i % nslot
            return pltpu.async_copy(
                table_hbm.at[idx_v.at[pl.ds(i * GW, GW)]],
                buf.at[slot],
                gsem.at[slot],
            )

        handles_g = [None] * ndma
        handles_o = [None] * ndma
        for i in range(min(2, ndma)):
            handles_g[i] = start_gather(i)
        for i in range(ndma):
            slot = i % nslot
            handles_g[i].wait()
            handles_o[i] = pltpu.async_copy(
                buf.at[slot], out_hbm.at[pl.ds(base + i * GW, GW)], osem.at[slot]
            )
            if i + 2 < ndma:
                if i - 1 >= 0:
                    handles_o[i - 1].wait()
                handles_g[i + 2] = start_gather(i + 2)
        for i in range(max(ndma - 2, 0), ndma):
            handles_o[i].wait()

    return k(word_emb, ids1d)


def _ln_body(g_ref, p_ref, w_ref, b_ref, o_ref):
    x = g_ref[...] + p_ref[...]
    inv_d = 1.0 / x.shape[1]
    u = jnp.sum(x, axis=1, keepdims=True) * inv_d
    s2 = jnp.sum(x * x, axis=1, keepdims=True) * inv_d
    v = s2 - u * u
    y = (x - u) * lax.rsqrt(v + EPS)
    y = y * w_ref[...] + b_ref[...]
    o_ref[0] = y.T


def _ln_body_acc(g_ref, p_ref, w_ref, b_ref, _buf_ref, o_ref):
    _ln_body(g_ref, p_ref, w_ref, b_ref, o_ref)


def _tc_chunk(gathered_c, pos_emb, w2d, b2d, buf, b_off, bpc, bsz, s, d):
    in_specs = [
        pl.BlockSpec((s, d), lambda b: (b, 0)),
        pl.BlockSpec((s, d), lambda b: (0, 0)),
        pl.BlockSpec((1, d), lambda b: (0, 0)),
        pl.BlockSpec((1, d), lambda b: (0, 0)),
    ]
    args = [gathered_c, pos_emb, w2d, b2d]
    if buf is None:
        body = _ln_body
        aliases = {}
    else:
        body = _ln_body_acc
        in_specs.append(pl.BlockSpec(memory_space=pl.ANY))
        args.append(buf)
        aliases = {4: 0}
    return pl.pallas_call(
        body,
        grid=(bpc,),
        in_specs=in_specs,
        out_specs=pl.BlockSpec(
            (1, d, s), lambda b, b_off=b_off: (b_off + b, 0, 0)
        ),
        out_shape=jax.ShapeDtypeStruct((bsz, d, s), jnp.float32),
        input_output_aliases=aliases,
        compiler_params=pltpu.CompilerParams(
            dimension_semantics=("arbitrary",),
        ),
    )(*args)


def kernel(input_ids, word_emb, pos_emb, ln_weight, ln_bias):
    bsz, s = input_ids.shape
    _, d = word_emb.shape
    bpc = bsz // NCHUNK
    ids1d = input_ids.astype(jnp.int32).reshape(bsz * s)
    w2d = ln_weight.reshape(1, d)
    b2d = ln_bias.reshape(1, d)
    gathered = [
        _sc_gather_chunk(word_emb, ids1d, c * bpc * s, bpc * s, d)
        for c in range(NCHUNK)
    ]
    buf = None
    for c in range(NCHUNK):
        buf = _tc_chunk(
            gathered[c], pos_emb, w2d, b2d, buf, c * bpc, bpc, bsz, s, d
        )
    return buf
